# Initial kernel scaffold; baseline (speedup 1.0000x reference)
import jax
import jax.numpy as jnp
from jax.experimental import pallas as pl


def kernel(x, edge_index, y, eps, labelEmb, W1, b1, Wmu, Wls, Wd1, bd1, Wd2, bd2):
    raise NotImplementedError("write your pallas kernel here")


# speedup vs baseline: 4.5068x; 4.5068x over previous
import functools

import jax
import jax.numpy as jnp
from jax import lax
from jax.experimental import pallas as pl
from jax.experimental.pallas import tpu as pltpu
from jax.experimental.pallas import tpu_sc as plsc

N = 10000
E = 320000
D = 128
LBL = 8
LAT = 16
NUM_LBL = 32
H = 128
DP = 144
W2 = 2 * LAT

NC, NS = 2, 16
NW = NC * NS
CH = 128
CPW = 80
NCHUNK = NW * CPW
EPAD = NCHUNK * CH
SLAB = 640
NPAD = NS * SLAB
DUMMY = NPAD - CH
BLK = 1000


def _seg_sum_parts(table, src2, dst2, width):
    mesh = plsc.VectorSubcoreMesh(core_axis_name="c", subcore_axis_name="s",
                                  num_cores=NC, num_subcores=NS)

    def body(table_hbm, src_hbm, dst_hbm, out_hbm, src_v, dst_v, rows_v, acc_sh, sem):
        c = lax.axis_index("c")
        s = lax.axis_index("s")
        wid = s * NC + c

        z16 = jnp.zeros((16,), jnp.float32)

        def zrow(r, carry):
            for cc in range(width // 16):
                rows_v[r, pl.ds(cc * 16, 16)] = z16
            return carry

        lax.fori_loop(0, CH, zrow, 0)

        def zslab(j, carry):
            pltpu.sync_copy(rows_v, acc_sh.at[pl.ds(s * SLAB + j * CH, CH)])
            return carry

        lax.fori_loop(0, SLAB // CH, zslab, 0)
        plsc.subcore_barrier()

        c0 = wid * CPW
        pltpu.sync_copy(src_hbm.at[pl.ds(c0, CPW)], src_v)
        pltpu.sync_copy(dst_hbm.at[pl.ds(c0, CPW)], dst_v)

        def step(j, carry):
            pltpu.async_copy(table_hbm.at[src_v.at[j]], rows_v, sem).wait()
            pltpu.sync_copy(rows_v, acc_sh.at[dst_v.at[j]], add=True)
            return carry

        lax.fori_loop(0, CPW, step, 0)
        plsc.subcore_barrier()

        def cout(j, carry):
            off = s * SLAB + j * CH
            pltpu.sync_copy(acc_sh.at[pl.ds(off, CH)], rows_v)
            pltpu.sync_copy(rows_v, out_hbm.at[c].at[pl.ds(off, CH)])
            return carry

        lax.fori_loop(0, SLAB // CH, cout, 0)

    run = pl.kernel(
        body,
        out_type=jax.ShapeDtypeStruct((NC, NPAD, width), jnp.float32),
        mesh=mesh,
        scratch_types=[
            pltpu.VMEM((CPW, CH), jnp.int32),
            pltpu.VMEM((CPW, CH), jnp.int32),
            pltpu.VMEM((CH, width), jnp.float32),
            pltpu.VMEM_SHARED((NPAD, width), jnp.float32),
            pltpu.SemaphoreType.DMA,
        ],
        compiler_params=pltpu.CompilerParams(use_tc_tiling_on_sc=False),
    )
    return run(table, src2, dst2)


def _enc_body(x_ref, y_ref, lab_ref, w1_ref, out_ref):
    yb = y_ref[0, 0, :]
    onehot = (yb[:, None] == lax.broadcasted_iota(jnp.int32, (BLK, NUM_LBL), 1))
    onehot = onehot.astype(jnp.float32)
    t2 = jnp.dot(lab_ref[...], w1_ref[D:, :], preferred_element_type=jnp.float32)
    u = jnp.dot(x_ref[...], w1_ref[:D, :], preferred_element_type=jnp.float32)
    u = u + jnp.dot(onehot, t2, preferred_element_type=jnp.float32)
    out_ref[:, :D] = u
    out_ref[:, D:] = jnp.ones((BLK, DP - D), jnp.float32)


def _mid_body(parts_ref, b1_ref, wcat_ref, out_ref):
    agg = parts_ref[0] + parts_ref[1]
    deg = jnp.clip(agg[:, D:D + 1], 1.0, None)
    h = jnp.maximum(agg[:, :D] / deg + b1_ref[0][None, :], 0.0)
    out_ref[...] = jnp.dot(h, wcat_ref[...], preferred_element_type=jnp.float32)


def _dec_body(p2_ref, pd_ref, eps_ref, y_ref, lab_ref, wd1_ref, bd1_ref,
              wd2_ref, bd2_ref, out_ref):
    agg2 = p2_ref[0] + p2_ref[1]
    deg = jnp.clip(pd_ref[0][:, D:D + 1] + pd_ref[1][:, D:D + 1], 1.0, None)
    hm = agg2 / deg
    mu = hm[:, :LAT]
    logstd = jnp.minimum(hm[:, LAT:], 10.0)
    z = mu + eps_ref[...] * jnp.exp(logstd)
    yb = y_ref[0, 0, :]
    onehot = (yb[:, None] == lax.broadcasted_iota(jnp.int32, (BLK, NUM_LBL), 1))
    onehot = onehot.astype(jnp.float32)
    t3 = jnp.dot(lab_ref[...], wd1_ref[LAT:, :], preferred_element_type=jnp.float32)
    d = (jnp.dot(z, wd1_ref[:LAT, :], preferred_element_type=jnp.float32)
         + jnp.dot(onehot, t3, preferred_element_type=jnp.float32)
         + bd1_ref[0][None, :])
    d = jnp.maximum(d, 0.0)
    out_ref[...] = (jnp.dot(d, wd2_ref[...], preferred_element_type=jnp.float32)
                    + bd2_ref[0][None, :])


def kernel(x, edge_index, y, eps, labelEmb, W1, b1, Wmu, Wls, Wd1, bd1, Wd2, bd2):
    grid = N // BLK
    y3 = y.reshape(grid, 1, BLK)
    pad = EPAD - E
    src2 = jnp.concatenate(
        [edge_index[0], jnp.zeros((pad,), jnp.int32)]).reshape(NCHUNK, CH)
    dst2 = jnp.concatenate(
        [edge_index[1], jnp.full((pad,), DUMMY, jnp.int32)]).reshape(NCHUNK, CH)

    u = pl.pallas_call(
        _enc_body,
        grid=(grid,),
        in_specs=[
            pl.BlockSpec((BLK, D), lambda i: (i, 0)),
            pl.BlockSpec((1, 1, BLK), lambda i: (i, 0, 0)),
            pl.BlockSpec((NUM_LBL, LBL), lambda i: (0, 0)),
            pl.BlockSpec((D + LBL, H), lambda i: (0, 0)),
        ],
        out_specs=pl.BlockSpec((BLK, DP), lambda i: (i, 0)),
        out_shape=jax.ShapeDtypeStruct((N, DP), jnp.float32),
    )(x, y3, labelEmb, W1)

    parts1 = _seg_sum_parts(u, src2, dst2, DP)

    wcat = jnp.concatenate([Wmu, Wls], axis=1)
    hw = pl.pallas_call(
        _mid_body,
        grid=(grid,),
        in_specs=[
            pl.BlockSpec((NC, BLK, DP), lambda i: (0, i, 0)),
            pl.BlockSpec((1, H), lambda i: (0, 0)),
            pl.BlockSpec((H, W2), lambda i: (0, 0)),
        ],
        out_specs=pl.BlockSpec((BLK, W2), lambda i: (i, 0)),
        out_shape=jax.ShapeDtypeStruct((N, W2), jnp.float32),
    )(parts1, b1.reshape(1, H), wcat)

    parts2 = _seg_sum_parts(hw, src2, dst2, W2)

    recon = pl.pallas_call(
        _dec_body,
        grid=(grid,),
        in_specs=[
            pl.BlockSpec((NC, BLK, W2), lambda i: (0, i, 0)),
            pl.BlockSpec((NC, BLK, DP), lambda i: (0, i, 0)),
            pl.BlockSpec((BLK, LAT), lambda i: (i, 0)),
            pl.BlockSpec((1, 1, BLK), lambda i: (i, 0, 0)),
            pl.BlockSpec((NUM_LBL, LBL), lambda i: (0, 0)),
            pl.BlockSpec((LAT + LBL, H), lambda i: (0, 0)),
            pl.BlockSpec((1, H), lambda i: (0, 0)),
            pl.BlockSpec((H, D), lambda i: (0, 0)),
            pl.BlockSpec((1, D), lambda i: (0, 0)),
        ],
        out_specs=pl.BlockSpec((BLK, D), lambda i: (i, 0)),
        out_shape=jax.ShapeDtypeStruct((N, D), jnp.float32),
    )(parts2, parts1, eps, y3, labelEmb, Wd1, bd1.reshape(1, H), Wd2,
      bd2.reshape(1, D))
    return recon
